# baseline (speedup 1.0000x reference)
import functools

import jax
import jax.numpy as jnp
from jax import lax
from jax.experimental import pallas as pl
from jax.experimental.pallas import tpu as pltpu
from jax.experimental.pallas import tpu_sc as plsc

_BATCH = 16384
_VOCAB = 1000000
_EMBED = 64
_NDENSE = 26
_NCOL = _NDENSE + 1
_OUT_D = _EMBED + _NDENSE

_INFO = plsc.get_sparse_core_info()
_NC = _INFO.num_cores
_NS = _INFO.num_subcores
_NW = _NC * _NS
_BPW = _BATCH // _NW
_RND = 64
_NRND = _BPW // _RND
_G = 16
_NG = _RND // _G


@functools.partial(
    pl.kernel,
    out_type=jax.ShapeDtypeStruct((_BATCH, _OUT_D), jnp.float32),
    mesh=plsc.VectorSubcoreMesh(core_axis_name="c", subcore_axis_name="s"),
    compiler_params=pltpu.CompilerParams(needs_layout_passes=False),
    scratch_types=[
        pltpu.VMEM((_RND, _NCOL), jnp.float32),
        pltpu.VMEM((_RND * 8, _EMBED), jnp.float32),
        pltpu.VMEM((2, _RND, _OUT_D), jnp.float32),
        pltpu.SemaphoreType.DMA,
        pltpu.SemaphoreType.DMA,
        pltpu.SemaphoreType.DMA,
        pltpu.SemaphoreType.DMA,
        pltpu.SemaphoreType.DMA,
        pltpu.SemaphoreType.DMA,
    ],
)
def _embed_sc(x_hbm, table_hbm, out_hbm, x_v, slab_v, out_v,
              sx, so, g0, g1, g2, g3):
    wid = lax.axis_index("s") * _NC + lax.axis_index("c")
    base = wid * _BPW
    gsems = (g0, g1, g2, g3)

    lanes = lax.iota(jnp.int32, 16)
    zvec = lanes * 0

    def round_body(j, carry):
        j64 = j * _RND
        p = j & 1
        for l in range(0, _RND, 8):
            pltpu.make_async_copy(
                x_hbm.at[pl.ds(base + j64 + l, 8)],
                x_v.at[pl.ds(l, 8)], sx).start()
        pltpu.make_async_copy(x_hbm.at[pl.ds(base, _RND)], x_v, sx).wait()
        rowvs = []
        for g in range(_NG):
            tvec = lanes + g * _G
            idv = plsc.load_gather(x_v, [tvec, zvec]).astype(jnp.int32)
            rowvs.append(tvec * 8 + (idv & 7))
            for l in range(_G):
                t = g * _G + l
                s8 = pl.multiple_of((idv[l] >> 3) * 8, 8)
                pltpu.make_async_copy(
                    table_hbm.at[pl.ds(s8, 8)],
                    slab_v.at[pl.ds(t * 8, 8)], gsems[g]).start()
        @pl.when(j > 0)
        def _():
            pltpu.make_async_copy(
                out_hbm.at[pl.ds(base, _RND)], out_v.at[1 - p], so).wait()
        pv = zvec + p
        for g in range(_NG):
            pltpu.make_async_copy(
                table_hbm.at[pl.ds(0, _G * 8)],
                slab_v.at[pl.ds(g * _G * 8, _G * 8)], gsems[g]).wait()
            tvec = lanes + g * _G
            rowv = rowvs[g]
            for c in range(_EMBED):
                val = plsc.load_gather(slab_v, [rowv, zvec + c])
                plsc.store_scatter(out_v, [pv, tvec, zvec + c], val)
            for c in range(_NDENSE):
                val = plsc.load_gather(x_v, [tvec, zvec + (c + 1)])
                plsc.store_scatter(out_v, [pv, tvec, zvec + (_EMBED + c)], val)
        for l in range(0, _RND, 8):
            pltpu.make_async_copy(
                out_v.at[p, pl.ds(l, 8)],
                out_hbm.at[pl.ds(base + j64 + l, 8)], so).start()
        return carry

    lax.fori_loop(0, _NRND, round_body, 0)
    pltpu.make_async_copy(
        out_hbm.at[pl.ds(base, _RND)], out_v.at[(_NRND - 1) & 1], so).wait()


def kernel(x, embed_weight):
    return _embed_sc(x, embed_weight)


# speedup vs baseline: 1.0751x; 1.0751x over previous
import functools

import jax
import jax.numpy as jnp
from jax import lax
from jax.experimental import pallas as pl
from jax.experimental.pallas import tpu as pltpu
from jax.experimental.pallas import tpu_sc as plsc

_BATCH = 16384
_VOCAB = 1000000
_EMBED = 64
_NDENSE = 26
_NCOL = 128
_OUT_D = _EMBED + _NDENSE

_INFO = plsc.get_sparse_core_info()
_NC = _INFO.num_cores
_NS = _INFO.num_subcores
_NW = _NC * _NS
_BPW = _BATCH // _NW
_RND = 64
_NRND = _BPW // _RND
_G = 16
_NG = _RND // _G


@functools.partial(
    pl.kernel,
    out_type=jax.ShapeDtypeStruct((_BATCH, 128), jnp.float32),
    mesh=plsc.VectorSubcoreMesh(core_axis_name="c", subcore_axis_name="s"),
    compiler_params=pltpu.CompilerParams(needs_layout_passes=False),
    scratch_types=[
        pltpu.VMEM((_RND, _NCOL), jnp.float32),
        pltpu.VMEM((_RND, 8, _EMBED), jnp.float32),
        pltpu.VMEM((_RND, 128), jnp.float32),
        pltpu.SemaphoreType.DMA,
        pltpu.SemaphoreType.DMA,
        pltpu.SemaphoreType.DMA,
        pltpu.SemaphoreType.DMA,
    ],
)
def _embed_sc(x_hbm, table_hbm, out_hbm, x_v, slab_v, out_v, s0, s1, s2, s3):
    wid = lax.axis_index("s") * _NC + lax.axis_index("c")
    base = wid * _BPW
    sems = (s0, s1, s2, s3)

    lanes = lax.iota(jnp.int32, 16)
    zvec = lanes * 0

    def round_body(j, carry):
        j64 = j * _RND
        pltpu.sync_copy(x_hbm.at[pl.ds(base + j64, _RND)], x_v)
        rvecs = []
        copies = []
        for g in range(_NG):
            tvec = lanes + g * _G
            idv = plsc.load_gather(x_v, [tvec, zvec]).astype(jnp.int32)
            rvecs.append(idv & 7)
            for l in range(_G):
                s8 = pl.multiple_of((idv[l] >> 3) * 8, 8)
                cp = pltpu.make_async_copy(
                    table_hbm.at[pl.ds(s8, 8)],
                    slab_v.at[g * _G + l],
                    sems[g],
                )
                cp.start()
                copies.append(cp)
        for g in range(_NG):
            for cp in copies[g * _G:(g + 1) * _G]:
                cp.wait()
            rvec = rvecs[g]
            for l in range(_G):
                t = g * _G + l
                r = rvec[l]
                for c in range(0, _EMBED, 16):
                    out_v[t, pl.ds(c, 16)] = slab_v[t, r, pl.ds(c, 16)]
                out_v[t, pl.ds(_EMBED, 16)] = x_v[t, pl.ds(1, 16)]
                out_v[t, pl.ds(_EMBED + 10, 16)] = x_v[t, pl.ds(11, 16)]
        pltpu.sync_copy(out_v, out_hbm.at[pl.ds(base + j64, _RND)])
        return carry

    lax.fori_loop(0, _NRND, round_body, 0)


def kernel(x, embed_weight):
    xp = jnp.pad(x, ((0, 0), (0, 128 - _NDENSE - 1)))
    return _embed_sc(xp, embed_weight)[:, :_OUT_D]


# speedup vs baseline: 1.0954x; 1.0188x over previous
import functools

import jax
import jax.numpy as jnp
from jax import lax
from jax.experimental import pallas as pl
from jax.experimental.pallas import tpu as pltpu
from jax.experimental.pallas import tpu_sc as plsc

_BATCH = 16384
_VOCAB = 1000000
_EMBED = 64
_NDENSE = 26
_NCOL = _NDENSE + 1
_OUT_D = _EMBED + _NDENSE

_INFO = plsc.get_sparse_core_info()
_NC = _INFO.num_cores
_NS = _INFO.num_subcores
_NW = _NC * _NS
_BPW = _BATCH // _NW
_RND = 64
_NRND = _BPW // _RND
_G = 16
_NG = _RND // _G


@functools.partial(
    pl.kernel,
    out_type=jax.ShapeDtypeStruct((_BATCH, _OUT_D), jnp.float32),
    mesh=plsc.VectorSubcoreMesh(core_axis_name="c", subcore_axis_name="s"),
    compiler_params=pltpu.CompilerParams(needs_layout_passes=False),
    scratch_types=[
        pltpu.VMEM((_RND, _NCOL), jnp.float32),
        pltpu.VMEM((_RND, 8, _EMBED), jnp.float32),
        pltpu.VMEM((_RND, _OUT_D), jnp.float32),
        pltpu.SemaphoreType.DMA,
        pltpu.SemaphoreType.DMA,
        pltpu.SemaphoreType.DMA,
        pltpu.SemaphoreType.DMA,
    ],
)
def _embed_sc(x_hbm, table_hbm, out_hbm, x_v, slab_v, out_v, s0, s1, s2, s3):
    wid = lax.axis_index("s") * _NC + lax.axis_index("c")
    base = wid * _BPW
    sems = (s0, s1, s2, s3)

    lanes = lax.iota(jnp.int32, 16)
    zvec = lanes * 0

    def round_body(j, carry):
        j64 = j * _RND
        pltpu.sync_copy(x_hbm.at[pl.ds(base + j64, _RND)], x_v)
        rvecs = []
        copies = []
        for g in range(_NG):
            tvec = lanes + g * _G
            idv = plsc.load_gather(x_v, [tvec, zvec]).astype(jnp.int32)
            rvecs.append(idv & 7)
            for l in range(_G):
                s8 = pl.multiple_of((idv[l] >> 3) * 8, 8)
                cp = pltpu.make_async_copy(
                    table_hbm.at[pl.ds(s8, 8)],
                    slab_v.at[g * _G + l],
                    sems[g],
                )
                cp.start()
                copies.append(cp)
        for g in range(_NG):
            for cp in copies[g * _G:(g + 1) * _G]:
                cp.wait()
            rvec = rvecs[g]
            for l in range(_G):
                t = g * _G + l
                r = rvec[l]
                for c in range(0, _EMBED, 16):
                    out_v[t, pl.ds(c, 16)] = slab_v[t, r, pl.ds(c, 16)]
                out_v[t, pl.ds(_EMBED, 16)] = x_v[t, pl.ds(1, 16)]
                out_v[t, pl.ds(_EMBED + 10, 16)] = x_v[t, pl.ds(11, 16)]
        pltpu.sync_copy(out_v, out_hbm.at[pl.ds(base + j64, _RND)])
        return carry

    lax.fori_loop(0, _NRND, round_body, 0)


def kernel(x, embed_weight):
    return _embed_sc(x, embed_weight)
